# baseline (speedup 1.0000x reference)
import functools

import jax
import jax.numpy as jnp
from jax import lax
from jax.experimental import pallas as pl
from jax.experimental.pallas import tpu as pltpu
from jax.experimental.pallas import tpu_sc as plsc

B, T, D = 2, 2048, 2048
NC, NS, L = 2, 16, 16
NW = NC * NS
ROWS_PER_W = T // NW
R = 8
CHUNK = R * D
N_CHUNKS = ROWS_PER_W // R
NBUF = 2

_mesh = plsc.VectorSubcoreMesh(core_axis_name="c", subcore_axis_name="s")


@functools.partial(
    pl.kernel,
    mesh=_mesh,
    out_type=jax.ShapeDtypeStruct((B * T * D,), jnp.float32),
    scratch_types=[
        pltpu.VMEM((NBUF, CHUNK), jnp.float32),
        pltpu.VMEM((NBUF, CHUNK), jnp.float32),
        pltpu.VMEM((NBUF, CHUNK), jnp.float32),
        pltpu.SemaphoreType.DMA,
        pltpu.SemaphoreType.DMA,
        pltpu.SemaphoreType.DMA,
        pltpu.SemaphoreType.DMA,
    ],
)
def _sc_add(x_hbm, pe_hbm, out_hbm, x0_v, x1_v, pe_v,
            in_sem0, in_sem1, out_sem0, out_sem1):
    in_sems = (in_sem0, in_sem1)
    out_sems = (out_sem0, out_sem1)
    wid = lax.axis_index("s") * NC + lax.axis_index("c")
    base = wid * (ROWS_PER_W * D)

    def issue_in(ci):
        s = ci % NBUF
        off = base + ci * CHUNK
        sem = in_sems[s]
        return (
            pltpu.async_copy(pe_hbm.at[pl.ds(off, CHUNK)], pe_v.at[s], sem),
            pltpu.async_copy(x_hbm.at[pl.ds(off, CHUNK)], x0_v.at[s], sem),
            pltpu.async_copy(x_hbm.at[pl.ds(T * D + off, CHUNK)], x1_v.at[s], sem),
        )

    def issue_out(ci):
        s = ci % NBUF
        off = base + ci * CHUNK
        sem = out_sems[s]
        return (
            pltpu.async_copy(x0_v.at[s], out_hbm.at[pl.ds(off, CHUNK)], sem),
            pltpu.async_copy(x1_v.at[s], out_hbm.at[pl.ds(T * D + off, CHUNK)], sem),
        )

    in_h = {0: issue_in(0)}
    out_h = {}

    for ci in range(N_CHUNKS):
        s = ci % NBUF
        nxt = ci + 1
        if nxt < N_CHUNKS:
            prev = nxt - NBUF
            if prev >= 0:
                for h in out_h.pop(prev):
                    h.wait()
            in_h[nxt] = issue_in(nxt)

        for h in in_h.pop(ci):
            h.wait()

        def add_body(i, c, s=s):
            pe16 = pe_v[s, pl.ds(i * L, L)]
            plsc.addupdate(x0_v.at[s, pl.ds(i * L, L)], pe16)
            plsc.addupdate(x1_v.at[s, pl.ds(i * L, L)], pe16)
            return c

        lax.fori_loop(0, CHUNK // L, add_body, 0, unroll=4)
        out_h[ci] = issue_out(ci)

    for ci in sorted(out_h):
        for h in out_h[ci]:
            h.wait()


def kernel(x, pe):
    out_flat = _sc_add(x.reshape(-1), pe.reshape(-1))
    return out_flat.reshape(B, T, D)


# speedup vs baseline: 1.1352x; 1.1352x over previous
import functools

import jax
import jax.numpy as jnp
from jax import lax
from jax.experimental import pallas as pl
from jax.experimental.pallas import tpu as pltpu
from jax.experimental.pallas import tpu_sc as plsc

B, T, D = 2, 2048, 2048
NC, NS, L = 2, 16, 16
NW = NC * NS
ROWS_PER_W = T // NW
R = 8
CHUNK = R * D
N_CHUNKS = ROWS_PER_W // R
NBUF = 2

_mesh = plsc.VectorSubcoreMesh(core_axis_name="c", subcore_axis_name="s")


@functools.partial(
    pl.kernel,
    mesh=_mesh,
    out_type=jax.ShapeDtypeStruct((B * T * D,), jnp.float32),
    scratch_types=[
        pltpu.VMEM((NBUF, CHUNK), jnp.float32),
        pltpu.VMEM((NBUF, CHUNK), jnp.float32),
        pltpu.VMEM((NBUF, CHUNK), jnp.float32),
        pltpu.SemaphoreType.DMA,
        pltpu.SemaphoreType.DMA,
        pltpu.SemaphoreType.DMA,
        pltpu.SemaphoreType.DMA,
    ],
)
def _sc_add(x_hbm, pe_hbm, out_hbm, x0_v, x1_v, pe_v,
            in_sem0, in_sem1, out_sem0, out_sem1):
    in_sems = (in_sem0, in_sem1)
    out_sems = (out_sem0, out_sem1)
    wid = lax.axis_index("s") * NC + lax.axis_index("c")
    base = wid * (ROWS_PER_W * D)

    def issue_in(ci):
        s = ci % NBUF
        off = base + ci * CHUNK
        sem = in_sems[s]
        return (
            pltpu.async_copy(pe_hbm.at[pl.ds(off, CHUNK)], pe_v.at[s], sem),
            pltpu.async_copy(x_hbm.at[pl.ds(off, CHUNK)], x0_v.at[s], sem),
            pltpu.async_copy(x_hbm.at[pl.ds(T * D + off, CHUNK)], x1_v.at[s], sem),
        )

    def issue_out(ci):
        s = ci % NBUF
        off = base + ci * CHUNK
        sem = out_sems[s]
        return (
            pltpu.async_copy(x0_v.at[s], out_hbm.at[pl.ds(off, CHUNK)], sem),
            pltpu.async_copy(x1_v.at[s], out_hbm.at[pl.ds(T * D + off, CHUNK)], sem),
        )

    in_h = {0: issue_in(0)}
    out_h = {}

    for ci in range(N_CHUNKS):
        s = ci % NBUF
        nxt = ci + 1
        if nxt < N_CHUNKS:
            prev = nxt - NBUF
            if prev >= 0:
                for h in out_h.pop(prev):
                    h.wait()
            in_h[nxt] = issue_in(nxt)

        for h in in_h.pop(ci):
            h.wait()

        @plsc.parallel_loop(0, CHUNK, step=L, unroll=8)
        def add_body(i, s=s):
            pe16 = pe_v[s, pl.ds(i, L)]
            plsc.addupdate(x0_v.at[s, pl.ds(i, L)], pe16)
            plsc.addupdate(x1_v.at[s, pl.ds(i, L)], pe16)
        out_h[ci] = issue_out(ci)

    for ci in sorted(out_h):
        for h in out_h[ci]:
            h.wait()


def kernel(x, pe):
    out_flat = _sc_add(x.reshape(-1), pe.reshape(-1))
    return out_flat.reshape(B, T, D)


# speedup vs baseline: 2.7215x; 2.3974x over previous
import functools

import jax
import jax.numpy as jnp
from jax import lax
from jax.experimental import pallas as pl
from jax.experimental.pallas import tpu as pltpu
from jax.experimental.pallas import tpu_sc as plsc

B, T, D = 2, 2048, 2048
NC, NS, L = 2, 16, 16
NW = NC * NS
ROWS_PER_W = T // NW
R = 8
N_CHUNKS = ROWS_PER_W // R
NBUF = 2

_mesh = plsc.VectorSubcoreMesh(core_axis_name="c", subcore_axis_name="s")


@functools.partial(
    pl.kernel,
    mesh=_mesh,
    out_type=jax.ShapeDtypeStruct((B * T, D), jnp.float32),
    scratch_types=[
        pltpu.VMEM((NBUF, R, D), jnp.float32),
        pltpu.VMEM((NBUF, R, D), jnp.float32),
        pltpu.VMEM((NBUF, R, D), jnp.float32),
        pltpu.SemaphoreType.DMA,
        pltpu.SemaphoreType.DMA,
        pltpu.SemaphoreType.DMA,
        pltpu.SemaphoreType.DMA,
    ],
)
def _sc_add(x_hbm, pe_hbm, out_hbm, x0_v, x1_v, pe_v,
            in_sem0, in_sem1, out_sem0, out_sem1):
    in_sems = (in_sem0, in_sem1)
    out_sems = (out_sem0, out_sem1)
    wid = lax.axis_index("s") * NC + lax.axis_index("c")
    base = wid * ROWS_PER_W

    def issue_in(ci):
        s = ci % NBUF
        row = base + ci * R
        sem = in_sems[s]
        return (
            pltpu.async_copy(pe_hbm.at[pl.ds(row, R), :], pe_v.at[s], sem),
            pltpu.async_copy(x_hbm.at[pl.ds(row, R), :], x0_v.at[s], sem),
            pltpu.async_copy(x_hbm.at[pl.ds(T + row, R), :], x1_v.at[s], sem),
        )

    def issue_out(ci):
        s = ci % NBUF
        row = base + ci * R
        sem = out_sems[s]
        return (
            pltpu.async_copy(x0_v.at[s], out_hbm.at[pl.ds(row, R), :], sem),
            pltpu.async_copy(x1_v.at[s], out_hbm.at[pl.ds(T + row, R), :], sem),
        )

    in_h = {0: issue_in(0)}
    out_h = {}

    for ci in range(N_CHUNKS):
        s = ci % NBUF
        nxt = ci + 1
        if nxt < N_CHUNKS:
            prev = nxt - NBUF
            if prev >= 0:
                for h in out_h.pop(prev):
                    h.wait()
            in_h[nxt] = issue_in(nxt)

        for h in in_h.pop(ci):
            h.wait()

        for r in range(R):
            @plsc.parallel_loop(0, D, step=L, unroll=8)
            def add_body(i, s=s, r=r):
                pe16 = pe_v[s, r, pl.ds(i, L)]
                plsc.addupdate(x0_v.at[s, r, pl.ds(i, L)], pe16)
                plsc.addupdate(x1_v.at[s, r, pl.ds(i, L)], pe16)

        out_h[ci] = issue_out(ci)

    for ci in sorted(out_h):
        for h in out_h[ci]:
            h.wait()


def kernel(x, pe):
    out2 = _sc_add(x.reshape(B * T, D), pe.reshape(T, D))
    return out2.reshape(B, T, D)


# speedup vs baseline: 2.9974x; 1.1014x over previous
import functools

import jax
import jax.numpy as jnp
from jax import lax
from jax.experimental import pallas as pl
from jax.experimental.pallas import tpu as pltpu
from jax.experimental.pallas import tpu_sc as plsc

B, T, D = 2, 2048, 2048
NC, NS, L = 2, 16, 16
NW = NC * NS
ROWS_PER_W = T // NW
R = 8
N_CHUNKS = ROWS_PER_W // R
NBUF = 2

_mesh = plsc.VectorSubcoreMesh(core_axis_name="c", subcore_axis_name="s")


@functools.partial(
    pl.kernel,
    mesh=_mesh,
    out_type=jax.ShapeDtypeStruct((B * T, D), jnp.float32),
    scratch_types=[
        pltpu.VMEM((NBUF, R, D), jnp.float32),
        pltpu.VMEM((NBUF, R, D), jnp.float32),
        pltpu.VMEM((NBUF, R, D), jnp.float32),
        pltpu.SemaphoreType.DMA,
        pltpu.SemaphoreType.DMA,
        pltpu.SemaphoreType.DMA,
        pltpu.SemaphoreType.DMA,
    ],
)
def _sc_add(x_hbm, pe_hbm, out_hbm, x0_v, x1_v, pe_v,
            in_sem0, in_sem1, out_sem0, out_sem1):
    in_sems = (in_sem0, in_sem1)
    out_sems = (out_sem0, out_sem1)
    wid = lax.axis_index("s") * NC + lax.axis_index("c")
    base = wid * ROWS_PER_W

    def issue_in(ci):
        s = ci % NBUF
        row = base + ci * R
        sem = in_sems[s]
        return (
            pltpu.async_copy(pe_hbm.at[pl.ds(row, R), :], pe_v.at[s], sem),
            pltpu.async_copy(x_hbm.at[pl.ds(row, R), :], x0_v.at[s], sem),
            pltpu.async_copy(x_hbm.at[pl.ds(T + row, R), :], x1_v.at[s], sem),
        )

    def issue_out(ci):
        s = ci % NBUF
        row = base + ci * R
        sem = out_sems[s]
        return (
            pltpu.async_copy(x0_v.at[s], out_hbm.at[pl.ds(row, R), :], sem),
            pltpu.async_copy(x1_v.at[s], out_hbm.at[pl.ds(T + row, R), :], sem),
        )

    in_h = {0: issue_in(0)}
    out_h = {}

    for ci in range(N_CHUNKS):
        s = ci % NBUF
        nxt = ci + 1
        if nxt < N_CHUNKS:
            prev = nxt - NBUF
            if prev >= 0:
                for h in out_h.pop(prev):
                    h.wait()
            in_h[nxt] = issue_in(nxt)

        for h in in_h.pop(ci):
            h.wait()

        @plsc.parallel_loop(0, D, step=L, unroll=2)
        def add_body(i, s=s):
            for r in range(R):
                pe16 = pe_v[s, r, pl.ds(i, L)]
                plsc.addupdate(x0_v.at[s, r, pl.ds(i, L)], pe16)
                plsc.addupdate(x1_v.at[s, r, pl.ds(i, L)], pe16)

        out_h[ci] = issue_out(ci)

    for ci in sorted(out_h):
        for h in out_h[ci]:
            h.wait()


def kernel(x, pe):
    out2 = _sc_add(x.reshape(B * T, D), pe.reshape(T, D))
    return out2.reshape(B, T, D)
